# baseline (speedup 1.0000x reference)
import functools

import jax
import jax.numpy as jnp
from jax import lax
from jax.experimental import pallas as pl
from jax.experimental.pallas import tpu as pltpu
from jax.experimental.pallas import tpu_sc as plsc

_NUM_EMBEDDINGS = 1000000
_EMBED_DIM = 64
_BATCH = 16384

_NC = 2
_NS = 16
_NW = _NC * _NS
_L = 16
_NTILE = 7813
_TPW = 245
_RING = 5
_STAGE = 128
_CAP = _BATCH + _L


def _lane0(vec):
    return lax.squeeze(lax.slice(vec, (0,), (1,)), (0,))


def _extract(vec, onehot):
    return jnp.sum(jnp.where(onehot, vec, 0))


@functools.partial(
    pl.kernel,
    mesh=plsc.VectorSubcoreMesh(core_axis_name="c", subcore_axis_name="s"),
    compiler_params=pltpu.CompilerParams(needs_layout_passes=False),
    out_type=jax.ShapeDtypeStruct((_BATCH + _NW, 2 * _EMBED_DIM),
                                  jnp.float32),
    scratch_types=[
        pltpu.VMEM((_BATCH,), jnp.int32),
        pltpu.VMEM((_CAP,), jnp.int32),
        pltpu.VMEM((_CAP,), jnp.int32),
        pltpu.VMEM((_RING, _EMBED_DIM, 128), jnp.float32),
        pltpu.VMEM((_STAGE, 2 * _EMBED_DIM), jnp.float32),
        pltpu.VMEM((_STAGE,), jnp.int32),
        pltpu.SemaphoreType.DMA,
        pltpu.SemaphoreType.DMA,
        pltpu.SemaphoreType.DMA,
        pltpu.SemaphoreType.DMA,
        pltpu.SemaphoreType.DMA,
        pltpu.SemaphoreType.DMA,
    ],
)
def _sweep_gather(idx_hbm, tab_hbm, out_hbm, idx_v, cand_i, cand_b,
                  bufs_v, stage_v, bvec_v, *sems):
    sem_f = sems[_RING]
    wid = lax.axis_index("s") * _NC + lax.axis_index("c")
    c_lo = (wid * (_NTILE - _TPW)) // (_NW - 1)
    iota = lax.iota(jnp.int32, _L)

    pltpu.sync_copy(idx_hbm, idx_v)
    dump = jnp.broadcast_to(_BATCH + wid, (_L,))
    for k in range(_STAGE // _L):
        bvec_v[pl.ds(k * _L, _L)] = dump

    lo = c_lo * 128
    hi = (c_lo + _TPW) * 128

    def _build(ch, cnt):
        v = idx_v[pl.ds(ch * _L, _L)]
        m = (v >= lo) & (v < hi)
        plsc.store_compressed(cand_i.at[pl.ds(cnt, _L)], v, mask=m)
        plsc.store_compressed(cand_b.at[pl.ds(cnt, _L)],
                              iota + ch * _L, mask=m)
        return cnt + _lane0(plsc.all_reduce_population_count(m))

    cnt = lax.fori_loop(0, _BATCH // _L, _build, jnp.int32(0))
    nq = (cnt + _L - 1) // _L

    def _fetch(c, slot):
        c0 = pl.multiple_of(c * 128, 128)
        pltpu.async_copy(
            tab_hbm.at[:, pl.ds(c0, 128)], bufs_v.at[slot], sems[slot])

    def _flush():
        pltpu.async_copy(stage_v, out_hbm.at[bvec_v], sem_f).wait()

    for r in range(_RING):
        _fetch(c_lo + r, r)

    def _tile_scan(c, slot, slot_cnt):
        buf = bufs_v.at[slot]

        def _q_body(q, sc):
            vi = cand_i[pl.ds(q * _L, _L)]
            vb = cand_b[pl.ds(q * _L, _L)]
            m0 = ((vi >> 7) == c) & (iota + q * _L < cnt)

            def _w_cond(carry):
                m, _ = carry
                return jnp.any(m)

            def _w_body(carry):
                m, sc2 = carry
                lane = _lane0(plsc.all_reduce_ffs(m))
                oh = iota == lane
                l = _extract(vi, oh) & 127
                b = _extract(vb, oh)
                col = jnp.broadcast_to(l, (_L,))
                row = jnp.broadcast_to(sc2, (_L,))
                for j in range(_EMBED_DIM // _L):
                    vals = plsc.load_gather(buf, [iota + j * _L, col])
                    plsc.store_scatter(stage_v, [row, iota + j * _L], vals)
                plsc.store_scatter(bvec_v, [row],
                                   jnp.broadcast_to(b, (_L,)),
                                   mask=iota == 0)
                sc3 = sc2 + 1

                def _full(_):
                    _flush()
                    return jnp.int32(0)

                sc4 = lax.cond(sc3 == _STAGE, _full, lambda x: x, sc3)
                return m & jnp.logical_not(oh), sc4

            _, sc_out = lax.while_loop(_w_cond, _w_body, (m0, sc))
            return sc_out

        return lax.fori_loop(0, nq, _q_body, slot_cnt)

    def _t5_body(t5, slot_cnt):
        sc = slot_cnt
        for r in range(_RING):
            c = c_lo + t5 * _RING + r
            pltpu.make_async_copy(
                tab_hbm.at[:, pl.ds(0, 128)], bufs_v.at[r], sems[r]
            ).wait()
            sc = _tile_scan(c, r, sc)

            @pl.when(t5 < _TPW // _RING - 1)
            def _():
                _fetch(c + _RING, r)
        return sc

    slot_cnt = lax.fori_loop(0, _TPW // _RING, _t5_body, jnp.int32(0))
    lax.cond(slot_cnt > 0, lambda: _flush(), lambda: None)


def kernel(input, table):
    idx = input.astype(jnp.int32)
    out_p = _sweep_gather(idx, table.T)
    return out_p[:_BATCH, :_EMBED_DIM].reshape(_BATCH, 1, 8, 8)


# speedup vs baseline: 1.8344x; 1.8344x over previous
import functools

import jax
import jax.numpy as jnp
from jax import lax
from jax.experimental import pallas as pl
from jax.experimental.pallas import tpu as pltpu
from jax.experimental.pallas import tpu_sc as plsc

_NUM_EMBEDDINGS = 1000000
_EMBED_DIM = 64
_BATCH = 16384

_NC = 2
_NS = 16
_NW = _NC * _NS
_L = 16
_NTILE = 7813
_TPW = 245
_RING = 4
_STAGE = 128
_CAP = _BATCH + _L
_HN = 272
_WN = _HN + 2 * _L


def _lane0(vec):
    return lax.squeeze(lax.slice(vec, (0,), (1,)), (0,))


def _lane15(vec):
    return lax.squeeze(lax.slice(vec, (_L - 1,), (_L,)), (0,))


def _extract(vec, onehot):
    return jnp.sum(jnp.where(onehot, vec, 0))


@functools.partial(
    pl.kernel,
    mesh=plsc.VectorSubcoreMesh(core_axis_name="c", subcore_axis_name="s"),
    compiler_params=pltpu.CompilerParams(needs_layout_passes=False),
    out_type=jax.ShapeDtypeStruct((_BATCH + _NW, 2 * _EMBED_DIM),
                                  jnp.float32),
    scratch_types=[
        pltpu.VMEM((_BATCH,), jnp.int32),
        pltpu.VMEM((_CAP,), jnp.int32),
        pltpu.VMEM((_CAP,), jnp.int32),
        pltpu.VMEM((_CAP,), jnp.int32),
        pltpu.VMEM((_HN,), jnp.int32),
        pltpu.VMEM((_HN,), jnp.int32),
        pltpu.VMEM((_HN,), jnp.int32),
        pltpu.VMEM((_WN,), jnp.int32),
        pltpu.VMEM((_RING, _EMBED_DIM, 128), jnp.float32),
        pltpu.VMEM((_STAGE, 2 * _EMBED_DIM), jnp.float32),
        pltpu.VMEM((_STAGE,), jnp.int32),
        pltpu.SemaphoreType.DMA,
        pltpu.SemaphoreType.DMA,
        pltpu.SemaphoreType.DMA,
        pltpu.SemaphoreType.DMA,
        pltpu.SemaphoreType.DMA,
    ],
)
def _sweep_gather(idx_hbm, tab_hbm, out_hbm, idx_v, raw_v, rawt_v, srt_v,
                  hist_v, starts_v, offs_v, wl_v, bufs_v, stage_v, bvec_v,
                  *sems):
    sem_f = sems[_RING]
    wid = lax.axis_index("s") * _NC + lax.axis_index("c")
    c_lo = (wid * (_NTILE - _TPW)) // (_NW - 1)
    iota = lax.iota(jnp.int32, _L)
    zeros = jnp.zeros((_L,), jnp.int32)
    lane0_m = iota == 0

    pltpu.sync_copy(idx_hbm, idx_v)
    dump = jnp.broadcast_to(_BATCH + wid, (_L,))
    for k in range(_STAGE // _L):
        bvec_v[pl.ds(k * _L, _L)] = dump
    for k in range(_HN // _L):
        hist_v[pl.ds(k * _L, _L)] = zeros

    def _sc_read(ref, pos):
        return _lane0(plsc.load_gather(ref, [jnp.broadcast_to(pos, (_L,))]))

    def _sc_write(ref, pos, val):
        plsc.store_scatter(ref, [jnp.broadcast_to(pos, (_L,))],
                           jnp.broadcast_to(val, (_L,)), mask=lane0_m)

    lo = c_lo * 128
    hi = (c_lo + _TPW) * 128

    def _build(ch, cnt):
        v = idx_v[pl.ds(ch * _L, _L)]
        m0 = (v >= lo) & (v < hi)

        def _cond(carry):
            return jnp.any(carry[0])

        def _body(carry):
            m, k = carry
            lane = _lane0(plsc.all_reduce_ffs(m))
            oh = iota == lane
            i = _extract(v, oh)
            t = (i >> 7) - c_lo
            _sc_write(raw_v, k, ((ch * _L + lane) << 7) | (i & 127))
            _sc_write(rawt_v, k, t)
            _sc_write(hist_v, t, _sc_read(hist_v, t) + 1)
            return m & jnp.logical_not(oh), k + 1

        _, cnt2 = lax.while_loop(_cond, _body, (m0, cnt))
        return cnt2

    cnt = lax.fori_loop(0, _BATCH // _L, _build, jnp.int32(0))

    def _prefix(k, carry):
        h = hist_v[pl.ds(k * _L, _L)]
        incl = plsc.cumsum(h) + carry
        excl = incl - h
        starts_v[pl.ds(k * _L, _L)] = excl
        offs_v[pl.ds(k * _L, _L)] = excl
        return _lane15(incl)

    lax.fori_loop(0, _HN // _L, _prefix, jnp.int32(0))

    def _place(q, carry):
        tq = rawt_v[pl.ds(q * _L, _L)]
        vq = raw_v[pl.ds(q * _L, _L)]
        for l in range(_L):
            @pl.when(q * _L + l < cnt)
            def _():
                oh = iota == l
                t = _extract(tq, oh)
                val = _extract(vq, oh)
                dest = _sc_read(offs_v, t)
                _sc_write(srt_v, dest, val)
                _sc_write(offs_v, t, dest + 1)
        return carry

    lax.fori_loop(0, (cnt + _L - 1) // _L, _place, jnp.int32(0))

    def _wl(k, n):
        h = hist_v[pl.ds(k * _L, _L)]
        m = h > 0
        plsc.store_compressed(wl_v.at[pl.ds(n, _L)], iota + k * _L, mask=m)
        return n + _lane0(plsc.all_reduce_population_count(m))

    n_act = lax.fori_loop(0, _HN // _L, _wl, jnp.int32(0))
    n_grp = (n_act + _RING - 1) // _RING
    n_pad = n_grp * _RING
    last = _sc_read(wl_v, jnp.maximum(n_act - 1, 0))

    @pl.when(n_act > 0)
    def _():
        plsc.store_scatter(
            wl_v, [n_act + iota], jnp.broadcast_to(last, (_L,)))

    def _fetch(t, slot):
        c0 = pl.multiple_of(t * 128, 128)
        pltpu.async_copy(
            tab_hbm.at[:, pl.ds(c0, 128)], bufs_v.at[slot], sems[slot])

    def _flush():
        pltpu.async_copy(stage_v, out_hbm.at[bvec_v], sem_f).wait()

    @pl.when(n_act > 0)
    def _():
        for r in range(_RING):
            _fetch(_sc_read(wl_v, jnp.minimum(r, n_pad - 1)) + c_lo, r)

    def _grp(g, slot_cnt):
        sc = slot_cnt
        for r in range(_RING):
            pos = g * _RING + r
            pltpu.make_async_copy(
                tab_hbm.at[:, pl.ds(0, 128)], bufs_v.at[r], sems[r]
            ).wait()
            t = _sc_read(wl_v, pos)
            seg_s = _sc_read(starts_v, t)
            seg_e = _sc_read(offs_v, t)
            buf = bufs_v.at[r]

            def _cand(k, sc2):
                val = _sc_read(srt_v, k)
                col = jnp.broadcast_to(val & 127, (_L,))
                row = jnp.broadcast_to(sc2, (_L,))
                for j in range(_EMBED_DIM // _L):
                    vals = plsc.load_gather(buf, [iota + j * _L, col])
                    plsc.store_scatter(stage_v, [row, iota + j * _L], vals)
                plsc.store_scatter(bvec_v, [row],
                                   jnp.broadcast_to(val >> 7, (_L,)),
                                   mask=lane0_m)
                sc3 = sc2 + 1

                def _full(_):
                    _flush()
                    return jnp.int32(0)

                return lax.cond(sc3 == _STAGE, _full, lambda x: x, sc3)

            sc = lax.fori_loop(seg_s, seg_e, _cand, sc)

            @pl.when(pos + _RING < n_pad)
            def _():
                _fetch(_sc_read(wl_v, pos + _RING) + c_lo, r)
        return sc

    slot_cnt = lax.fori_loop(0, n_grp, _grp, jnp.int32(0))
    lax.cond(slot_cnt > 0, lambda: _flush(), lambda: None)


def kernel(input, table):
    idx = input.astype(jnp.int32)
    out_p = _sweep_gather(idx, table.T)
    return out_p[:_BATCH, :_EMBED_DIM].reshape(_BATCH, 1, 8, 8)


# speedup vs baseline: 2.3081x; 1.2582x over previous
import functools

import jax
import jax.numpy as jnp
from jax import lax
from jax.experimental import pallas as pl
from jax.experimental.pallas import tpu as pltpu
from jax.experimental.pallas import tpu_sc as plsc

_NUM_EMBEDDINGS = 1000000
_EMBED_DIM = 64
_BATCH = 16384

_NC = 2
_NS = 16
_NW = _NC * _NS
_L = 16
_NTILE = 7813
_TPW = 245
_RING = 4
_STAGE = 128
_CAP = _BATCH + _L
_HN = 272
_WN = _HN + 2 * _L


def _lane0(vec):
    return lax.squeeze(lax.slice(vec, (0,), (1,)), (0,))


def _lane15(vec):
    return lax.squeeze(lax.slice(vec, (_L - 1,), (_L,)), (0,))


def _extract(vec, onehot):
    return jnp.sum(jnp.where(onehot, vec, 0))


@functools.partial(
    pl.kernel,
    mesh=plsc.VectorSubcoreMesh(core_axis_name="c", subcore_axis_name="s"),
    compiler_params=pltpu.CompilerParams(needs_layout_passes=False),
    out_type=jax.ShapeDtypeStruct((_BATCH + _NW, 2 * _EMBED_DIM),
                                  jnp.float32),
    scratch_types=[
        pltpu.VMEM((_BATCH,), jnp.int32),
        pltpu.VMEM((_CAP,), jnp.int32),
        pltpu.VMEM((_CAP,), jnp.int32),
        pltpu.VMEM((_CAP,), jnp.int32),
        pltpu.VMEM((_HN,), jnp.int32),
        pltpu.VMEM((_HN,), jnp.int32),
        pltpu.VMEM((_HN,), jnp.int32),
        pltpu.VMEM((_WN,), jnp.int32),
        pltpu.VMEM((_RING, _EMBED_DIM, 128), jnp.float32),
        pltpu.VMEM((_STAGE, 2 * _EMBED_DIM), jnp.float32),
        pltpu.VMEM((_STAGE,), jnp.int32),
        pltpu.SemaphoreType.DMA,
        pltpu.SemaphoreType.DMA,
        pltpu.SemaphoreType.DMA,
        pltpu.SemaphoreType.DMA,
        pltpu.SemaphoreType.DMA,
    ],
)
def _sweep_gather(idx_hbm, tab_hbm, out_hbm, idx_v, raw_v, rawt_v, srt_v,
                  hist_v, starts_v, offs_v, wl_v, bufs_v, stage_v, bvec_v,
                  *sems):
    sem_f = sems[_RING]
    wid = lax.axis_index("s") * _NC + lax.axis_index("c")
    c_lo = (wid * (_NTILE - _TPW)) // (_NW - 1)
    iota = lax.iota(jnp.int32, _L)
    zeros = jnp.zeros((_L,), jnp.int32)
    lane0_m = iota == 0

    pltpu.sync_copy(idx_hbm, idx_v)
    dump = jnp.broadcast_to(_BATCH + wid, (_L,))
    for k in range(_STAGE // _L):
        bvec_v[pl.ds(k * _L, _L)] = dump
    for k in range(_HN // _L):
        hist_v[pl.ds(k * _L, _L)] = zeros

    def _sc_read(ref, pos):
        return _lane0(plsc.load_gather(ref, [jnp.broadcast_to(pos, (_L,))]))

    def _sc_write(ref, pos, val):
        plsc.store_scatter(ref, [jnp.broadcast_to(pos, (_L,))],
                           jnp.broadcast_to(val, (_L,)), mask=lane0_m)

    lo = c_lo * 128
    hi = (c_lo + _TPW) * 128

    ones = jnp.ones((_L,), jnp.int32)

    def _build(ch, cnt):
        v = idx_v[pl.ds(ch * _L, _L)]
        m = (v >= lo) & (v < hi)
        t = (v >> 7) - c_lo
        raw = ((iota + ch * _L) << 7) | (v & 127)
        plsc.store_compressed(raw_v.at[pl.ds(cnt, _L)], raw, mask=m)
        plsc.store_compressed(rawt_v.at[pl.ds(cnt, _L)], t, mask=m)
        plsc.addupdate_scatter(hist_v, [jnp.where(m, t, 0)], ones,
                               mask=m)
        return cnt + _lane0(plsc.all_reduce_population_count(m))

    cnt = lax.fori_loop(0, _BATCH // _L, _build, jnp.int32(0))

    def _prefix(k, carry):
        h = hist_v[pl.ds(k * _L, _L)]
        incl = plsc.cumsum(h) + carry
        excl = incl - h
        starts_v[pl.ds(k * _L, _L)] = excl
        offs_v[pl.ds(k * _L, _L)] = excl
        return _lane15(incl)

    lax.fori_loop(0, _HN // _L, _prefix, jnp.int32(0))

    def _wl(k, n):
        h = hist_v[pl.ds(k * _L, _L)]
        m = h > 0
        plsc.store_compressed(wl_v.at[pl.ds(n, _L)], iota + k * _L, mask=m)
        return n + _lane0(plsc.all_reduce_population_count(m))

    n_act = lax.fori_loop(0, _HN // _L, _wl, jnp.int32(0))
    n_grp = (n_act + _RING - 1) // _RING
    n_pad = n_grp * _RING
    last = _sc_read(wl_v, jnp.maximum(n_act - 1, 0))

    @pl.when(n_act > 0)
    def _():
        plsc.store_scatter(
            wl_v, [n_act + iota], jnp.broadcast_to(last, (_L,)))

    def _fetch(t, slot):
        c0 = pl.multiple_of(t * 128, 128)
        pltpu.async_copy(
            tab_hbm.at[:, pl.ds(c0, 128)], bufs_v.at[slot], sems[slot])

    def _flush():
        pltpu.async_copy(stage_v, out_hbm.at[bvec_v], sem_f).wait()

    @pl.when(n_act > 0)
    def _():
        for r in range(_RING):
            _fetch(_sc_read(wl_v, jnp.minimum(r, n_pad - 1)) + c_lo, r)

    def _place(q, carry):
        tq = rawt_v[pl.ds(q * _L, _L)]
        vq = raw_v[pl.ds(q * _L, _L)]
        for l in range(_L):
            @pl.when(q * _L + l < cnt)
            def _():
                oh = iota == l
                t = _extract(tq, oh)
                val = _extract(vq, oh)
                dest = _sc_read(offs_v, t)
                _sc_write(srt_v, dest, val)
                _sc_write(offs_v, t, dest + 1)
        return carry

    lax.fori_loop(0, (cnt + _L - 1) // _L, _place, jnp.int32(0))

    def _grp(g, slot_cnt):
        sc = slot_cnt
        for r in range(_RING):
            pos = g * _RING + r
            pltpu.make_async_copy(
                tab_hbm.at[:, pl.ds(0, 128)], bufs_v.at[r], sems[r]
            ).wait()
            t = _sc_read(wl_v, pos)
            seg_s = _sc_read(starts_v, t)
            seg_e = _sc_read(offs_v, t)
            buf = bufs_v.at[r]

            def _cand(k, sc2):
                val = _sc_read(srt_v, k)
                col = jnp.broadcast_to(val & 127, (_L,))
                row = jnp.broadcast_to(sc2, (_L,))
                for j in range(_EMBED_DIM // _L):
                    vals = plsc.load_gather(buf, [iota + j * _L, col])
                    plsc.store_scatter(stage_v, [row, iota + j * _L], vals)
                plsc.store_scatter(bvec_v, [row],
                                   jnp.broadcast_to(val >> 7, (_L,)),
                                   mask=lane0_m)
                sc3 = sc2 + 1

                def _full(_):
                    _flush()
                    return jnp.int32(0)

                return lax.cond(sc3 == _STAGE, _full, lambda x: x, sc3)

            sc = lax.fori_loop(seg_s, seg_e, _cand, sc)

            @pl.when(pos + _RING < n_pad)
            def _():
                _fetch(_sc_read(wl_v, pos + _RING) + c_lo, r)
        return sc

    slot_cnt = lax.fori_loop(0, n_grp, _grp, jnp.int32(0))
    lax.cond(slot_cnt > 0, lambda: _flush(), lambda: None)


def kernel(input, table):
    idx = input.astype(jnp.int32)
    out_p = _sweep_gather(idx, table.T)
    return out_p[:_BATCH, :_EMBED_DIM].reshape(_BATCH, 1, 8, 8)


# speedup vs baseline: 2.6090x; 1.1304x over previous
import functools

import jax
import jax.numpy as jnp
from jax import lax
from jax.experimental import pallas as pl
from jax.experimental.pallas import tpu as pltpu
from jax.experimental.pallas import tpu_sc as plsc

_NUM_EMBEDDINGS = 1000000
_EMBED_DIM = 64
_BATCH = 16384

_NC = 2
_NS = 16
_NW = _NC * _NS
_L = 16
_NTILE = 7813
_TPW = 245
_RING = 5
_STAGE = 128
_CAP = _BATCH + _L
_HN = 272
_WN = _HN + 2 * _L


def _lane0(vec):
    return lax.squeeze(lax.slice(vec, (0,), (1,)), (0,))


def _lane15(vec):
    return lax.squeeze(lax.slice(vec, (_L - 1,), (_L,)), (0,))


def _extract(vec, onehot):
    return jnp.sum(jnp.where(onehot, vec, 0))


@functools.partial(
    pl.kernel,
    mesh=plsc.VectorSubcoreMesh(core_axis_name="c", subcore_axis_name="s"),
    compiler_params=pltpu.CompilerParams(needs_layout_passes=False),
    out_type=jax.ShapeDtypeStruct((_BATCH + _NW, 2 * _EMBED_DIM),
                                  jnp.float32),
    scratch_types=[
        pltpu.VMEM((_BATCH,), jnp.int32),
        pltpu.VMEM((_CAP,), jnp.int32),
        pltpu.VMEM((_CAP,), jnp.int32),
        pltpu.VMEM((_HN,), jnp.int32),
        pltpu.VMEM((_HN,), jnp.int32),
        pltpu.VMEM((_HN,), jnp.int32),
        pltpu.VMEM((_WN,), jnp.int32),
        pltpu.VMEM((_RING, _EMBED_DIM, 128), jnp.float32),
        pltpu.VMEM((_STAGE, 2 * _EMBED_DIM), jnp.float32),
        pltpu.VMEM((_STAGE,), jnp.int32),
        pltpu.SemaphoreType.DMA,
        pltpu.SemaphoreType.DMA,
        pltpu.SemaphoreType.DMA,
        pltpu.SemaphoreType.DMA,
        pltpu.SemaphoreType.DMA,
        pltpu.SemaphoreType.DMA,
    ],
)
def _sweep_gather(idx_hbm, tab_hbm, out_hbm, idx_v, raw_v, srt_v,
                  hist_v, starts_v, offs_v, wl_v, bufs_v, stage_v, bvec_v,
                  *sems):
    sem_f = sems[_RING]
    wid = lax.axis_index("s") * _NC + lax.axis_index("c")
    c_lo = (wid * (_NTILE - _TPW)) // (_NW - 1)
    iota = lax.iota(jnp.int32, _L)
    zeros = jnp.zeros((_L,), jnp.int32)
    lane0_m = iota == 0

    pltpu.sync_copy(idx_hbm, idx_v)
    dump = jnp.broadcast_to(_BATCH + wid, (_L,))
    for k in range(_STAGE // _L):
        bvec_v[pl.ds(k * _L, _L)] = dump
    for k in range(_HN // _L):
        hist_v[pl.ds(k * _L, _L)] = zeros

    def _sc_read(ref, pos):
        return _lane0(plsc.load_gather(ref, [jnp.broadcast_to(pos, (_L,))]))

    def _sc_write(ref, pos, val):
        plsc.store_scatter(ref, [jnp.broadcast_to(pos, (_L,))],
                           jnp.broadcast_to(val, (_L,)), mask=lane0_m)

    lo = c_lo * 128
    hi = (c_lo + _TPW) * 128

    ones = jnp.ones((_L,), jnp.int32)

    def _build(ch, cnt):
        v = idx_v[pl.ds(ch * _L, _L)]
        m = (v >= lo) & (v < hi)
        t = (v >> 7) - c_lo
        raw = (t << 21) | ((iota + ch * _L) << 7) | (v & 127)
        plsc.store_compressed(raw_v.at[pl.ds(cnt, _L)], raw, mask=m)
        plsc.addupdate_scatter(hist_v, [jnp.where(m, t, 0)], ones,
                               mask=m)
        return cnt + _lane0(plsc.all_reduce_population_count(m))

    cnt = lax.fori_loop(0, _BATCH // _L, _build, jnp.int32(0))

    def _prefix(k, carry):
        h = hist_v[pl.ds(k * _L, _L)]
        incl = plsc.cumsum(h) + carry
        excl = incl - h
        starts_v[pl.ds(k * _L, _L)] = excl
        offs_v[pl.ds(k * _L, _L)] = excl
        return _lane15(incl)

    lax.fori_loop(0, _HN // _L, _prefix, jnp.int32(0))

    def _wl(k, n):
        h = hist_v[pl.ds(k * _L, _L)]
        m = h > 0
        plsc.store_compressed(wl_v.at[pl.ds(n, _L)], iota + k * _L, mask=m)
        return n + _lane0(plsc.all_reduce_population_count(m))

    n_act = lax.fori_loop(0, _HN // _L, _wl, jnp.int32(0))
    n_grp = (n_act + _RING - 1) // _RING
    n_pad = n_grp * _RING
    last = _sc_read(wl_v, jnp.maximum(n_act - 1, 0))

    @pl.when(n_act > 0)
    def _():
        plsc.store_scatter(
            wl_v, [n_act + iota], jnp.broadcast_to(last, (_L,)))

    def _fetch(t, slot):
        c0 = pl.multiple_of(t * 128, 128)
        pltpu.async_copy(
            tab_hbm.at[:, pl.ds(c0, 128)], bufs_v.at[slot], sems[slot])

    def _flush():
        pltpu.async_copy(stage_v, out_hbm.at[bvec_v], sem_f).wait()

    @pl.when(n_act > 0)
    def _():
        for r in range(_RING):
            _fetch(_sc_read(wl_v, jnp.minimum(r, n_pad - 1)) + c_lo, r)

    def _place(q, carry):
        vq = raw_v[pl.ds(q * _L, _L)]
        tq = vq >> 21
        mt = iota + q * _L < cnt
        rank, _last = plsc.scan_count(tq, mask=mt)
        bases = plsc.load_gather(offs_v, [jnp.where(mt, tq, 0)])
        dest = bases + rank - 1
        plsc.store_scatter(srt_v, [jnp.where(mt, dest, 0)], vq, mask=mt)
        plsc.store_scatter(offs_v, [jnp.where(mt, tq, 0)], dest + 1,
                           mask=mt & _last)
        return carry

    lax.fori_loop(0, (cnt + _L - 1) // _L, _place, jnp.int32(0))

    def _grp(g, slot_cnt):
        sc = slot_cnt
        for r in range(_RING):
            pos = g * _RING + r
            pltpu.make_async_copy(
                tab_hbm.at[:, pl.ds(0, 128)], bufs_v.at[r], sems[r]
            ).wait()
            t = _sc_read(wl_v, pos)
            seg_s = _sc_read(starts_v, t)
            seg_e = _sc_read(offs_v, t)
            buf = bufs_v.at[r]

            def _cand(k, sc2):
                val = _sc_read(srt_v, k)
                col = jnp.broadcast_to(val & 127, (_L,))
                row = jnp.broadcast_to(sc2, (_L,))
                for j in range(_EMBED_DIM // _L):
                    vals = plsc.load_gather(buf, [iota + j * _L, col])
                    plsc.store_scatter(stage_v, [row, iota + j * _L], vals)
                plsc.store_scatter(bvec_v, [row],
                                   jnp.broadcast_to((val >> 7) & 16383, (_L,)),
                                   mask=lane0_m)
                sc3 = sc2 + 1

                def _full(_):
                    _flush()
                    return jnp.int32(0)

                return lax.cond(sc3 == _STAGE, _full, lambda x: x, sc3)

            sc = lax.fori_loop(seg_s, seg_e, _cand, sc)

            @pl.when(pos + _RING < n_pad)
            def _():
                _fetch(_sc_read(wl_v, pos + _RING) + c_lo, r)
        return sc

    slot_cnt = lax.fori_loop(0, n_grp, _grp, jnp.int32(0))
    lax.cond(slot_cnt > 0, lambda: _flush(), lambda: None)


def kernel(input, table):
    idx = input.astype(jnp.int32)
    out_p = _sweep_gather(idx, table.T)
    return out_p[:_BATCH, :_EMBED_DIM].reshape(_BATCH, 1, 8, 8)
